# Initial kernel scaffold; baseline (speedup 1.0000x reference)
import jax
import jax.numpy as jnp
from jax.experimental import pallas as pl


def kernel(pos, edge_index, W1, b1, W2, b2, W3, b3):
    raise NotImplementedError("write your pallas kernel here")


# speedup vs baseline: 4.2376x; 4.2376x over previous
import jax
import jax.numpy as jnp
from jax import lax
from jax.experimental import pallas as pl
from jax.experimental.pallas import tpu as pltpu
from jax.experimental.pallas import tpu_sc as plsc

N = 10000
E = 320000
C = 128
NP = 10240
TRASH = N
NC_SC = 2
NS_SC = 16
NW = NC_SC * NS_SC
CE = 128
NCH = 79
EW = NCH * CE
E_PAD = NW * EW
SL = NP // NS_SC
BN = 512
GRID = NP // BN


def _sc_segsum_body(gidx, sidx, table, zeros, out, acc, gbuf, sbuf, rows, sem):
    _I0 = jnp.int32(0)
    cid = lax.axis_index("c")
    sid = lax.axis_index("s")
    wid = cid * jnp.int32(NS_SC) + sid
    row0 = sid * jnp.int32(SL)
    pltpu.sync_copy(zeros.at[pl.ds(row0, SL)], acc.at[pl.ds(row0, SL)])
    plsc.subcore_barrier()
    base = wid * jnp.int32(EW)

    def body(c, carry):
        off = base + c * jnp.int32(CE)
        pltpu.sync_copy(gidx.at[pl.ds(off, CE)], gbuf.at[_I0])
        pltpu.sync_copy(sidx.at[pl.ds(off, CE)], sbuf.at[_I0])
        pltpu.async_copy(table.at[gbuf.at[_I0]], rows, sem).wait()
        pltpu.sync_copy(rows, acc.at[sbuf.at[_I0]], add=True)
        return carry

    lax.fori_loop(jnp.int32(0), jnp.int32(NCH), body, jnp.int32(0))
    plsc.subcore_barrier()
    pltpu.sync_copy(acc.at[pl.ds(row0, SL)], out.at[cid, pl.ds(row0, SL)])


_sc_segsum = pl.kernel(
    _sc_segsum_body,
    out_type=jax.ShapeDtypeStruct((NC_SC, NP, C), jnp.float32),
    mesh=plsc.VectorSubcoreMesh(core_axis_name="c", subcore_axis_name="s"),
    scratch_types=[
        pltpu.VMEM_SHARED((NP, C), jnp.float32),
        pltpu.VMEM((1, CE), jnp.int32),
        pltpu.VMEM((1, CE), jnp.int32),
        pltpu.VMEM((CE, C), jnp.float32),
        pltpu.SemaphoreType.DMA,
    ],
)


def _dinv_body(p, dv_out):
    pv = p[...]
    deg = pv[0, :, 0:1] + pv[1, :, 0:1]
    dv_out[...] = jnp.where(deg > 0, 1.0 / jnp.sqrt(jnp.maximum(deg, 1.0)), 0.0)


_tc_dinv = pl.pallas_call(
    _dinv_body,
    grid=(GRID,),
    in_specs=[pl.BlockSpec((NC_SC, BN, C), lambda i: (jnp.int32(0), i, jnp.int32(0)))],
    out_specs=pl.BlockSpec((BN, 1), lambda i: (i, jnp.int32(0))),
    out_shape=jax.ShapeDtypeStruct((NP, 1), jnp.float32),
)


def _pre_body(x, dinv, w, g_out, o_out):
    xv = x[...]
    g_out[...] = dinv[...] * xv
    o_out[...] = jnp.dot(xv, w[...], preferred_element_type=jnp.float32)


_tc_pre = pl.pallas_call(
    _pre_body,
    grid=(GRID,),
    in_specs=[
        pl.BlockSpec((BN, C), lambda i: (i, jnp.int32(0))),
        pl.BlockSpec((BN, 1), lambda i: (i, jnp.int32(0))),
        pl.BlockSpec((C, C), lambda i: (jnp.int32(0), jnp.int32(0))),
    ],
    out_specs=[pl.BlockSpec((BN, C), lambda i: (i, jnp.int32(0)))] * 2,
    out_shape=[jax.ShapeDtypeStruct((NP, C), jnp.float32)] * 2,
)


def _make_combine(first, last, residual):
    def body(*refs):
        i = 0
        p = refs[i]; i += 1
        dinv = refs[i]; i += 1
        txp2 = None
        if not first:
            txp2 = refs[i]; i += 1
        oin = refs[i]; i += 1
        w = refs[i]; i += 1
        b = res = None
        if last:
            b = refs[i]; i += 1
            if residual:
                res = refs[i]; i += 1
        outs = refs[i:]

        pv = p[...]
        s = pv[0] + pv[1]
        dv = dinv[...]
        if first:
            tx = -(dv * s)
        else:
            tx = -2.0 * (dv * s) - txp2[...]
        o = oin[...] + jnp.dot(tx, w[...], preferred_element_type=jnp.float32)
        if last:
            h = jnp.maximum(o + b[...], 0.0)
            if residual:
                h = h + res[...]
            outs[0][...] = h
        else:
            outs[0][...] = tx
            outs[1][...] = dv * tx
            outs[2][...] = o

    row = pl.BlockSpec((BN, C), lambda i: (i, jnp.int32(0)))
    in_specs = [pl.BlockSpec((NC_SC, BN, C), lambda i: (jnp.int32(0), i, jnp.int32(0))),
                pl.BlockSpec((BN, 1), lambda i: (i, jnp.int32(0)))]
    if not first:
        in_specs.append(row)
    in_specs.append(row)
    in_specs.append(pl.BlockSpec((C, C), lambda i: (jnp.int32(0), jnp.int32(0))))
    if last:
        in_specs.append(pl.BlockSpec((1, C), lambda i: (jnp.int32(0), jnp.int32(0))))
        if residual:
            in_specs.append(row)
        out_specs = [row]
        out_shape = [jax.ShapeDtypeStruct((NP, C), jnp.float32)]
    else:
        out_specs = [row] * 3
        out_shape = [jax.ShapeDtypeStruct((NP, C), jnp.float32)] * 3

    return pl.pallas_call(
        body, grid=(GRID,), in_specs=in_specs, out_specs=out_specs,
        out_shape=out_shape)


_tc_first = _make_combine(first=True, last=False, residual=False)
_tc_mid = _make_combine(first=False, last=False, residual=False)
_tc_last = _make_combine(first=False, last=True, residual=False)
_tc_last_res = _make_combine(first=False, last=True, residual=True)


def _layer(x, dinv, W, b, res, gidx, sidx, zeros):
    g, out = _tc_pre(x, dinv, W[0])
    p = _sc_segsum(gidx, sidx, g, zeros)
    tx1, g, out = _tc_first(p, dinv, out, W[1])
    txm2, txm1 = x, tx1
    for k in (2, 3, 4):
        p = _sc_segsum(gidx, sidx, g, zeros)
        txk, g, out = _tc_mid(p, dinv, txm2, out, W[k])
        txm2, txm1 = txm1, txk
    p = _sc_segsum(gidx, sidx, g, zeros)
    if res is None:
        (h,) = _tc_last(p, dinv, txm2, out, W[5], b)
    else:
        (h,) = _tc_last_res(p, dinv, txm2, out, W[5], b, res)
    return h


def kernel(pos, edge_index, W1, b1, W2, b2, W3, b3):
    pos = pos.astype(jnp.float32)
    src = edge_index[0].astype(jnp.int32)
    dst = edge_index[1].astype(jnp.int32)
    selfe = src == dst
    npad = E_PAD - E
    gidx = jnp.concatenate([src, jnp.zeros((npad,), jnp.int32)])
    sidx_lap = jnp.concatenate(
        [jnp.where(selfe, TRASH, dst), jnp.full((npad,), TRASH, jnp.int32)])
    sidx_deg = jnp.concatenate(
        [jnp.where(selfe, TRASH, src), jnp.full((npad,), TRASH, jnp.int32)])

    zeros = jnp.zeros((NP, C), jnp.float32)
    ones = jnp.ones((NP, C), jnp.float32)
    x = jnp.zeros((NP, C), jnp.float32).at[:N].set(pos)

    pdeg = _sc_segsum(gidx, sidx_deg, ones, zeros)
    dinv = _tc_dinv(pdeg)

    h = x
    for W, b, res in ((W1, b1, None), (W2, b2, None), (W3, b3, x)):
        h = _layer(h, dinv, W.astype(jnp.float32),
                   b.astype(jnp.float32).reshape(1, C), res, gidx, sidx_lap,
                   zeros)
    return h[:N]
